# baseline (speedup 1.0000x reference)
import functools

import jax
import jax.numpy as jnp
from jax.experimental import pallas as pl

TOPK = 8
NUM_EXPERTS = 64
ROW_BLOCK = 4096


def _router_block(x_ref, w_ref, b_ref, probs_ref, idx_ref):
    x = x_ref[...]
    w = w_ref[...]
    scores = jnp.dot(x, w, preferred_element_type=jnp.float32) + b_ref[...]

    rows = scores.shape[0]
    iota = jax.lax.broadcasted_iota(jnp.int32, (rows, NUM_EXPERTS), 1).astype(
        jnp.float32)
    vals = scores
    top_vals = []
    top_idx = []
    for k in range(TOPK):
        m = jnp.max(vals, axis=1, keepdims=True)
        idx = jnp.min(jnp.where(vals == m, iota, float(NUM_EXPERTS)), axis=1,
                      keepdims=True)
        top_vals.append(m)
        top_idx.append(idx)
        if k + 1 < TOPK:
            vals = jnp.where(iota == idx, -jnp.inf, vals)

    lane8 = jax.lax.broadcasted_iota(jnp.int32, (rows, TOPK), 1)
    v = top_vals[0]
    ix = top_idx[0]
    for k in range(1, TOPK):
        sel = lane8 == k
        v = jnp.where(sel, top_vals[k], v)
        ix = jnp.where(sel, top_idx[k], ix)
    e = jnp.exp(v - top_vals[0])
    probs_ref[...] = e / jnp.sum(e, axis=1, keepdims=True)
    idx_ref[...] = ix.astype(jnp.int32)


@jax.jit
def kernel(inputs, W, b):
    n_rows = inputs.shape[0]
    grid = (n_rows // ROW_BLOCK,)
    probs, idx = pl.pallas_call(
        _router_block,
        grid=grid,
        in_specs=[
            pl.BlockSpec((ROW_BLOCK, inputs.shape[1]), lambda i: (i, 0)),
            pl.BlockSpec((inputs.shape[1], NUM_EXPERTS), lambda i: (0, 0)),
            pl.BlockSpec((1, NUM_EXPERTS), lambda i: (0, 0)),
        ],
        out_specs=[
            pl.BlockSpec((ROW_BLOCK, TOPK), lambda i: (i, 0)),
            pl.BlockSpec((ROW_BLOCK, TOPK), lambda i: (i, 0)),
        ],
        out_shape=[
            jax.ShapeDtypeStruct((n_rows, TOPK), jnp.float32),
            jax.ShapeDtypeStruct((n_rows, TOPK), jnp.int32),
        ],
    )(inputs, W, b.reshape(1, NUM_EXPERTS))
    return probs, idx
